# baseline (speedup 1.0000x reference)
import functools

import jax
import jax.numpy as jnp
from jax import lax
from jax.experimental import pallas as pl
from jax.experimental.pallas import tpu as pltpu
from jax.experimental.pallas import tpu_sc as plsc

BQ = 512
BQC = 2048
BIGF = 1e30
SC_CHUNK = 512
CPAD = 128


def _knn_block(pos_skip_ref, pos_t_ref, idx_ref, w_ref):
    M = pos_t_ref.shape[1]
    q = pos_skip_ref[...]
    d2 = jnp.zeros((q.shape[0], M), jnp.float32)
    for c in range(3):
        d = q[:, c:c + 1] - pos_t_ref[c:c + 1, :]
        d2 = d2 + d * d
    m0 = jnp.min(d2, axis=1, keepdims=True)
    work = jnp.where(d2 <= m0, BIGF, d2)
    m1 = jnp.min(work, axis=1, keepdims=True)
    work = jnp.where(d2 <= m1, BIGF, d2)
    m2 = jnp.min(work, axis=1, keepdims=True)
    iota = lax.broadcasted_iota(jnp.int32, (M, 1), 0)
    rhs = jnp.concatenate([(iota // 64).astype(jnp.float32),
                           (iota % 64).astype(jnp.float32)], axis=1)
    s1 = jnp.dot(jnp.where(d2 <= m0, 1.0, 0.0), rhs,
                 preferred_element_type=jnp.float32)
    s2 = jnp.dot(jnp.where(d2 <= m1, 1.0, 0.0), rhs,
                 preferred_element_type=jnp.float32)
    s3 = jnp.dot(jnp.where(d2 <= m2, 1.0, 0.0), rhs,
                 preferred_element_type=jnp.float32)
    c0 = s1
    c1 = s2 - s1
    c2 = s3 - s2
    idx = jnp.concatenate([64.0 * c0[:, 0:1] + c0[:, 1:2],
                           64.0 * c1[:, 0:1] + c1[:, 1:2],
                           64.0 * c2[:, 0:1] + c2[:, 1:2]], axis=1)
    idx_ref[...] = jnp.clip(idx + 0.5, 0.0, float(M - 1)).astype(jnp.int32)
    v0 = 1.0 / (m0 + 1e-8)
    v1 = 1.0 / (m1 + 1e-8)
    v2 = 1.0 / (m2 + 1e-8)
    inv = 1.0 / (v0 + v1 + v2 + 1e-8)
    w_ref[...] = jnp.concatenate([v0, v1, v2], axis=1) * inv


def _combine_block(w_ref, g0_ref, g1_ref, g2_ref, xs_ref, w1a_ref, w1b_ref,
                   b1_ref, out_ref):
    w = w_ref[...]
    nc = w1a_ref.shape[0]
    xi = (w[:, 0:1] * g0_ref[:, :nc] + w[:, 1:2] * g1_ref[:, :nc]
          + w[:, 2:3] * g2_ref[:, :nc])
    h = jnp.dot(xi, w1a_ref[...], preferred_element_type=jnp.float32)
    h = h + jnp.dot(xs_ref[...], w1b_ref[...],
                    preferred_element_type=jnp.float32)
    h = h + b1_ref[...]
    out_ref[...] = jnp.maximum(h, 0.0)


def _sc_gather(x_hbm, idx_hbm, out_hbm, idx_v, rows_v, sem):
    info = plsc.get_sparse_core_info()
    nw = info.num_cores * info.num_subcores
    wid = lax.axis_index("s") * info.num_cores + lax.axis_index("c")
    total = out_hbm.shape[0]
    per_w = total // nw
    for j in range(per_w // SC_CHUNK):
        base = wid * per_w + j * SC_CHUNK
        pltpu.sync_copy(idx_hbm.at[pl.ds(base, SC_CHUNK)], idx_v)
        pltpu.async_copy(x_hbm.at[idx_v], rows_v, sem).wait()
        pltpu.sync_copy(rows_v, out_hbm.at[pl.ds(base, SC_CHUNK)])


def kernel(x, pos, batch, x_skip, pos_skip, batch_skip, W1, b1):
    M, C = x.shape
    N, Cs = x_skip.shape
    H = W1.shape[1]
    pos_t = pos.T
    w1a = W1[:C]
    w1b = W1[C:]
    b1r = b1.reshape(1, H)

    idx, w = pl.pallas_call(
        _knn_block,
        grid=(N // BQ,),
        in_specs=[
            pl.BlockSpec((BQ, 3), lambda i: (i, 0)),
            pl.BlockSpec((3, M), lambda i: (0, 0)),
        ],
        out_specs=[
            pl.BlockSpec((BQ, 3), lambda i: (i, 0)),
            pl.BlockSpec((BQ, 3), lambda i: (i, 0)),
        ],
        out_shape=[
            jax.ShapeDtypeStruct((N, 3), jnp.int32),
            jax.ShapeDtypeStruct((N, 3), jnp.float32),
        ],
        compiler_params=pltpu.CompilerParams(
            dimension_semantics=("arbitrary",)),
    )(pos_skip, pos_t)

    idx_flat = idx.T.reshape(3 * N)

    x_pad = jnp.pad(x, ((0, 0), (0, CPAD - C)))
    mesh = plsc.VectorSubcoreMesh(core_axis_name="c", subcore_axis_name="s")
    gathered = pl.kernel(
        _sc_gather,
        mesh=mesh,
        out_type=jax.ShapeDtypeStruct((3 * N, CPAD), jnp.float32),
        scratch_types=[
            pltpu.VMEM((SC_CHUNK,), jnp.int32),
            pltpu.VMEM((SC_CHUNK, CPAD), jnp.float32),
            pltpu.SemaphoreType.DMA,
        ],
        compiler_params=pltpu.CompilerParams(use_tc_tiling_on_sc=False),
    )(x_pad, idx_flat)

    nb = N // BQC
    return pl.pallas_call(
        _combine_block,
        grid=(nb,),
        in_specs=[
            pl.BlockSpec((BQC, 3), lambda i: (i, 0)),
            pl.BlockSpec((BQC, CPAD), lambda i: (i, 0)),
            pl.BlockSpec((BQC, CPAD), lambda i: (i + nb, 0)),
            pl.BlockSpec((BQC, CPAD), lambda i: (i + 2 * nb, 0)),
            pl.BlockSpec((BQC, Cs), lambda i: (i, 0)),
            pl.BlockSpec((C, H), lambda i: (0, 0)),
            pl.BlockSpec((Cs, H), lambda i: (0, 0)),
            pl.BlockSpec((1, H), lambda i: (0, 0)),
        ],
        out_specs=pl.BlockSpec((BQC, H), lambda i: (i, 0)),
        out_shape=jax.ShapeDtypeStruct((N, H), jnp.float32),
        compiler_params=pltpu.CompilerParams(
            dimension_semantics=("arbitrary",)),
    )(w, gathered, gathered, gathered, x_skip, w1a, w1b, b1r)


# speedup vs baseline: 1.0650x; 1.0650x over previous
import functools

import jax
import jax.numpy as jnp
from jax import lax
from jax.experimental import pallas as pl
from jax.experimental.pallas import tpu as pltpu
from jax.experimental.pallas import tpu_sc as plsc

BQ = 512
BQC = 2048
BIGF = 1e30
SC_CHUNK = 512
CPAD = 128


def _knn_block(pos_skip_ref, pos_t_ref, idx_ref, w_ref):
    M = pos_t_ref.shape[1]
    q = pos_skip_ref[...]
    d2 = jnp.zeros((q.shape[0], M), jnp.float32)
    for c in range(3):
        d = q[:, c:c + 1] - pos_t_ref[c:c + 1, :]
        d2 = d2 + d * d
    m0 = jnp.min(d2, axis=1, keepdims=True)
    work = jnp.where(d2 <= m0, BIGF, d2)
    m1 = jnp.min(work, axis=1, keepdims=True)
    work = jnp.where(work <= m1, BIGF, work)
    m2 = jnp.min(work, axis=1, keepdims=True)
    iota = lax.broadcasted_iota(jnp.int32, (M, 1), 0)
    rhs = jnp.concatenate([(iota // 64).astype(jnp.float32),
                           (iota % 64).astype(jnp.float32)], axis=1)
    s1 = jnp.dot(jnp.where(d2 <= m0, 1.0, 0.0), rhs,
                 preferred_element_type=jnp.float32)
    s2 = jnp.dot(jnp.where(d2 <= m1, 1.0, 0.0), rhs,
                 preferred_element_type=jnp.float32)
    s3 = jnp.dot(jnp.where(d2 <= m2, 1.0, 0.0), rhs,
                 preferred_element_type=jnp.float32)
    c0 = s1
    c1 = s2 - s1
    c2 = s3 - s2
    idx = jnp.concatenate([64.0 * c0[:, 0:1] + c0[:, 1:2],
                           64.0 * c1[:, 0:1] + c1[:, 1:2],
                           64.0 * c2[:, 0:1] + c2[:, 1:2]], axis=1)
    idx_ref[...] = jnp.clip(idx + 0.5, 0.0, float(M - 1)).astype(jnp.int32)
    v0 = 1.0 / (m0 + 1e-8)
    v1 = 1.0 / (m1 + 1e-8)
    v2 = 1.0 / (m2 + 1e-8)
    inv = 1.0 / (v0 + v1 + v2 + 1e-8)
    w_ref[...] = jnp.concatenate([v0, v1, v2], axis=1) * inv


def _combine_block(w_ref, g0_ref, g1_ref, g2_ref, xs_ref, w1a_ref, w1b_ref,
                   b1_ref, out_ref):
    w = w_ref[...]
    nc = w1a_ref.shape[0]
    xi = (w[:, 0:1] * g0_ref[:, :nc] + w[:, 1:2] * g1_ref[:, :nc]
          + w[:, 2:3] * g2_ref[:, :nc])
    h = jnp.dot(xi, w1a_ref[...], preferred_element_type=jnp.float32)
    h = h + jnp.dot(xs_ref[...], w1b_ref[...],
                    preferred_element_type=jnp.float32)
    h = h + b1_ref[...]
    out_ref[...] = jnp.maximum(h, 0.0)


def _sc_gather(x_hbm, idx_hbm, out_hbm, idx_v, rows_v, sem):
    info = plsc.get_sparse_core_info()
    nw = info.num_cores * info.num_subcores
    wid = lax.axis_index("s") * info.num_cores + lax.axis_index("c")
    total = out_hbm.shape[0]
    per_w = total // nw
    for j in range(per_w // SC_CHUNK):
        base = wid * per_w + j * SC_CHUNK
        pltpu.sync_copy(idx_hbm.at[pl.ds(base, SC_CHUNK)], idx_v)
        pltpu.async_copy(x_hbm.at[idx_v], rows_v, sem).wait()
        pltpu.sync_copy(rows_v, out_hbm.at[pl.ds(base, SC_CHUNK)])


def kernel(x, pos, batch, x_skip, pos_skip, batch_skip, W1, b1):
    M, C = x.shape
    N, Cs = x_skip.shape
    H = W1.shape[1]
    pos_t = pos.T
    w1a = W1[:C]
    w1b = W1[C:]
    b1r = b1.reshape(1, H)

    idx, w = pl.pallas_call(
        _knn_block,
        grid=(N // BQ,),
        in_specs=[
            pl.BlockSpec((BQ, 3), lambda i: (i, 0)),
            pl.BlockSpec((3, M), lambda i: (0, 0)),
        ],
        out_specs=[
            pl.BlockSpec((BQ, 3), lambda i: (i, 0)),
            pl.BlockSpec((BQ, 3), lambda i: (i, 0)),
        ],
        out_shape=[
            jax.ShapeDtypeStruct((N, 3), jnp.int32),
            jax.ShapeDtypeStruct((N, 3), jnp.float32),
        ],
        compiler_params=pltpu.CompilerParams(
            dimension_semantics=("arbitrary",)),
    )(pos_skip, pos_t)

    idx_flat = idx.T.reshape(3 * N)

    x_pad = jnp.pad(x, ((0, 0), (0, CPAD - C)))
    mesh = plsc.VectorSubcoreMesh(core_axis_name="c", subcore_axis_name="s")
    gathered = pl.kernel(
        _sc_gather,
        mesh=mesh,
        out_type=jax.ShapeDtypeStruct((3 * N, CPAD), jnp.float32),
        scratch_types=[
            pltpu.VMEM((SC_CHUNK,), jnp.int32),
            pltpu.VMEM((SC_CHUNK, CPAD), jnp.float32),
            pltpu.SemaphoreType.DMA,
        ],
        compiler_params=pltpu.CompilerParams(use_tc_tiling_on_sc=False),
    )(x_pad, idx_flat)

    nb = N // BQC
    return pl.pallas_call(
        _combine_block,
        grid=(nb,),
        in_specs=[
            pl.BlockSpec((BQC, 3), lambda i: (i, 0)),
            pl.BlockSpec((BQC, CPAD), lambda i: (i, 0)),
            pl.BlockSpec((BQC, CPAD), lambda i: (i + nb, 0)),
            pl.BlockSpec((BQC, CPAD), lambda i: (i + 2 * nb, 0)),
            pl.BlockSpec((BQC, Cs), lambda i: (i, 0)),
            pl.BlockSpec((C, H), lambda i: (0, 0)),
            pl.BlockSpec((Cs, H), lambda i: (0, 0)),
            pl.BlockSpec((1, H), lambda i: (0, 0)),
        ],
        out_specs=pl.BlockSpec((BQC, H), lambda i: (i, 0)),
        out_shape=jax.ShapeDtypeStruct((N, H), jnp.float32),
        compiler_params=pltpu.CompilerParams(
            dimension_semantics=("arbitrary",)),
    )(w, gathered, gathered, gathered, x_skip, w1a, w1b, b1r)
